# baseline (speedup 1.0000x reference)
import functools

import jax
import jax.numpy as jnp
from jax import lax
from jax.experimental import pallas as pl
from jax.experimental.pallas import tpu as pltpu
from jax.experimental.pallas import tpu_sc as plsc

N = 10000
E = 320000
D = 128
NCLS = 10
NC, NS = 2, 16
NW = NC * NS
NPAD = 10240
B = 128
EPT_R = E // NW
CPT_F = EPT_R // B
TAIL = EPT_R - CPT_F * B
TAILPAD = B - TAIL
CPT = CPT_F + 1
B2 = 64
CPT2F = EPT_R // B2
TAIL2 = EPT_R - CPT2F * B2
RPS_INIT = 632
RPS_OUT = NPAD // NS

_MESH = dict(core_axis_name="c", subcore_axis_name="s",
             num_cores=NC, num_subcores=NS)


def _fill_idx(didx_v, dibuf, i):
    for j in range(B // 16):
        didx_v[pl.ds(j * 16, 16)] = dibuf[pl.ds(i * B + j * 16, 16)]


def _deg_body(ei_hbm, out_hbm, ones_v, zeros_v, didx0, didx1, dibuf, acc,
              semd0, semd1):
    c = lax.axis_index("c")
    s = lax.axis_index("s")
    wid = s * NC + c
    base = wid * EPT_R
    for i in range(B // 16):
        ones_v[pl.ds(i * 16, 16)] = jnp.full((16,), 1.0, jnp.float32)
        zeros_v[pl.ds(i * 16, 16)] = jnp.zeros((16,), jnp.float32)
    for i in range(RPS_OUT // B):
        pltpu.sync_copy(zeros_v, acc.at[pl.ds(s * RPS_OUT + i * B, B)])
    pltpu.sync_copy(ei_hbm.at[1, wid], dibuf)
    plsc.subcore_barrier()

    def body(k, carry):
        i0 = 2 * k
        i1 = i0 + 1
        _fill_idx(didx0, dibuf, i0)
        pltpu.async_copy(ones_v, acc.at[didx0], semd0, add=True)
        _fill_idx(didx1, dibuf, i1)
        pltpu.async_copy(ones_v, acc.at[didx1], semd1, add=True)
        pltpu.make_async_copy(ones_v, acc.at[didx0], semd0).wait()
        pltpu.make_async_copy(ones_v, acc.at[didx1], semd1).wait()
        return carry

    lax.fori_loop(0, (CPT - 1) // 2, body, 0)
    didx0[pl.ds(0, 16)] = dibuf[pl.ds(EPT_R - TAIL, 16)]
    for j in range(1, B // 16):
        didx0[pl.ds(j * 16, 16)] = (
            lax.iota(jnp.int32, 16) + (N + (j - 1) * 16))
    pltpu.sync_copy(ones_v, acc.at[didx0], add=True)
    plsc.subcore_barrier()
    pltpu.sync_copy(acc.at[pl.ds(s * RPS_OUT, RPS_OUT)],
                    out_hbm.at[c, pl.ds(s * RPS_OUT, RPS_OUT)])


@functools.cache
def _deg_call():
    return pl.kernel(
        _deg_body,
        out_type=jax.ShapeDtypeStruct((NC, NPAD), jnp.float32),
        mesh=plsc.VectorSubcoreMesh(**_MESH),
        scratch_types=[
            pltpu.VMEM((B,), jnp.float32),
            pltpu.VMEM((B,), jnp.float32),
            pltpu.VMEM((B,), jnp.int32),
            pltpu.VMEM((B,), jnp.int32),
            pltpu.VMEM((EPT_R,), jnp.int32),
            pltpu.VMEM_SHARED((NPAD,), jnp.float32),
            pltpu.SemaphoreType.DMA,
            pltpu.SemaphoreType.DMA,
        ],
    )


def _agg_body(hp_hbm, ei_hbm, zeros_hbm, out_hbm,
              sibuf, sidx_t, didx_t, didx_t16, didxp0, didxp1,
              didx0, didx1, didx2, didx3, rows0, rows1, rows2, rows3, acc,
              semg0, semg1, semg2, semg3, sempd0, sempd1,
              sems0, sems1, sems2, sems3):
    c = lax.axis_index("c")
    s = lax.axis_index("s")
    wid = s * NC + c

    @pl.when(c == 0)
    def _():
        b0 = jnp.minimum(s * RPS_INIT, N - RPS_INIT)
        pltpu.sync_copy(hp_hbm.at[pl.ds(b0, RPS_INIT)],
                        acc.at[pl.ds(b0, RPS_INIT)])

    @pl.when(c == 1)
    def _():
        pltpu.sync_copy(zeros_hbm, acc.at[pl.ds(s * RPS_OUT, RPS_OUT)])

    pltpu.sync_copy(ei_hbm.at[0, wid], sibuf)
    sidx_t[pl.ds(0, 16)] = sibuf[pl.ds(EPT_R - TAIL2, 16)]
    for j in range(1, B2 // 16):
        sidx_t[pl.ds(j * 16, 16)] = (lax.iota(jnp.int32, 16) + j * 16) * 97
    for j in range(1, B2 // 16):
        didx_t[pl.ds(j * 16, 16)] = (
            lax.iota(jnp.int32, 16) + (N + (j - 1) * 16))
    plsc.subcore_barrier()

    didxs = (didx0, didx1, didx2, didx3)
    rows = (rows0, rows1, rows2, rows3)
    semg = (semg0, semg1, semg2, semg3)
    pairs = (didxp0, didxp1)
    sempd = (sempd0, sempd1)
    sems = (sems0, sems1, sems2, sems3)

    def gsrc(i):
        return hp_hbm.at[sibuf.at[pl.ds(i * B2, B2)]]

    def psrc(m):
        return ei_hbm.at[1, wid, pl.ds(m * B2, 2 * B2)]

    def fire_pair(m, pj):
        pltpu.async_copy(psrc(m), pairs[pj], sempd[pj])

    def wait_pair(m, pj):
        pltpu.make_async_copy(psrc(m), pairs[pj], sempd[pj]).wait()

    def copy_half(j, pj, half):
        for t in range(B2 // 16):
            didxs[j][pl.ds(t * 16, 16)] = (
                pairs[pj][pl.ds(half * B2 + t * 16, 16)])

    def fire_g(i, j):
        pltpu.async_copy(gsrc(i), rows[j], semg[j])

    def wait_g(i, j):
        pltpu.make_async_copy(gsrc(i), rows[j], semg[j]).wait()

    def fire_s(j):
        pltpu.async_copy(rows[j], acc.at[didxs[j]], sems[j], add=True)

    def wait_s(j):
        pltpu.make_async_copy(rows[j], acc.at[didxs[j]], sems[j]).wait()

    fire_pair(0, 0)
    fire_g(0, 0)
    fire_g(1, 1)
    fire_pair(2, 1)
    wait_g(0, 0)
    wait_pair(0, 0)
    copy_half(0, 0, 0)
    fire_s(0)
    fire_g(2, 2)
    wait_g(1, 1)
    copy_half(1, 0, 1)
    fire_s(1)
    fire_g(3, 3)

    def body(k, carry):
        m = 4 * k + 2
        wait_g(m, 2)
        wait_pair(m, 1)
        copy_half(2, 1, 0)
        fire_s(2)
        wait_s(0)
        fire_pair(m + 2, 0)
        fire_g(m + 2, 0)
        wait_g(m + 1, 3)
        copy_half(3, 1, 1)
        fire_s(3)
        wait_s(1)
        fire_g(m + 3, 1)
        wait_g(m + 2, 0)
        wait_pair(m + 2, 0)
        copy_half(0, 0, 0)
        fire_s(0)
        wait_s(2)
        fire_pair(m + 4, 1)
        fire_g(m + 4, 2)
        wait_g(m + 3, 1)
        copy_half(1, 0, 1)
        fire_s(1)
        wait_s(3)
        fire_g(m + 5, 3)
        return carry

    lax.fori_loop(0, (CPT2F - 4) // 4, body, 0)
    m0 = CPT2F - 2
    wait_g(m0, 2)
    wait_pair(m0, 1)
    copy_half(2, 1, 0)
    fire_s(2)
    wait_s(0)
    wait_g(m0 + 1, 3)
    copy_half(3, 1, 1)
    fire_s(3)
    wait_s(1)
    pltpu.async_copy(hp_hbm.at[sidx_t], rows0, semg0)
    tail_dsrc = ei_hbm.at[1, wid, pl.ds(EPT_R - TAIL2, TAIL2)]
    pltpu.async_copy(tail_dsrc, didx_t16, sempd0)
    pltpu.make_async_copy(hp_hbm.at[sidx_t], rows0, semg0).wait()
    pltpu.make_async_copy(tail_dsrc, didx_t16, sempd0).wait()
    didx_t[pl.ds(0, TAIL2)] = didx_t16[...]
    pltpu.sync_copy(rows0, acc.at[didx_t], add=True)
    wait_s(2)
    wait_s(3)
    plsc.subcore_barrier()
    pltpu.sync_copy(acc.at[pl.ds(s * RPS_OUT, RPS_OUT)],
                    out_hbm.at[c, pl.ds(s * RPS_OUT, RPS_OUT)])


@functools.cache
def _agg_call():
    return pl.kernel(
        _agg_body,
        out_type=jax.ShapeDtypeStruct((NC, NPAD, D), jnp.float32),
        mesh=plsc.VectorSubcoreMesh(**_MESH),
        scratch_types=[
            pltpu.VMEM((EPT_R,), jnp.int32),
            pltpu.VMEM((B2,), jnp.int32),
            pltpu.VMEM((B2,), jnp.int32),
            pltpu.VMEM((TAIL2,), jnp.int32),
            pltpu.VMEM((2 * B2,), jnp.int32),
            pltpu.VMEM((2 * B2,), jnp.int32),
            pltpu.VMEM((B2,), jnp.int32),
            pltpu.VMEM((B2,), jnp.int32),
            pltpu.VMEM((B2,), jnp.int32),
            pltpu.VMEM((B2,), jnp.int32),
            pltpu.VMEM((B2, D), jnp.float32),
            pltpu.VMEM((B2, D), jnp.float32),
            pltpu.VMEM((B2, D), jnp.float32),
            pltpu.VMEM((B2, D), jnp.float32),
            pltpu.VMEM_SHARED((NPAD, D), jnp.float32),
        ] + [pltpu.SemaphoreType.DMA] * 10,
    )


RB = 2000


def _dis(d0_ref, d1_ref):
    return lax.rsqrt(d0_ref[0] + d1_ref[0] + 1.0)


def _tc1_body(x_ref, w_ref, d0_ref, d1_ref, o_ref):
    h = jnp.dot(x_ref[...], w_ref[...], preferred_element_type=jnp.float32)
    o_ref[...] = h * _dis(d0_ref, d1_ref)


def _tc2_body(a0_ref, a1_ref, d0_ref, d1_ref, b_ref, w_ref, o_ref):
    dis = _dis(d0_ref, d1_ref)
    h1 = jnp.maximum((a0_ref[0] + a1_ref[0]) * dis + b_ref[...], 0.0)
    o_ref[...] = jnp.dot(h1, w_ref[...], preferred_element_type=jnp.float32) * dis


def _tc3_body(a0_ref, a1_ref, d0_ref, d1_ref, b_ref, w_ref, bc_ref, o_ref):
    dis = _dis(d0_ref, d1_ref)
    h2 = jnp.maximum((a0_ref[0] + a1_ref[0]) * dis + b_ref[...], 0.0)
    o_ref[...] = (jnp.dot(h2, w_ref[...], preferred_element_type=jnp.float32)
                  + bc_ref[...])


def _row_spec(width):
    return pl.BlockSpec((RB, width), lambda i: (i, 0))


def _part_spec(core, width):
    return pl.BlockSpec((1, RB, width), lambda i, _c=core: (_c, i, 0))


def _full_spec(shape):
    return pl.BlockSpec(shape, lambda i: tuple(0 for _ in shape))


_tc1 = pl.pallas_call(
    _tc1_body,
    grid=(N // RB,),
    in_specs=[_row_spec(D), _full_spec((D, D)),
              _part_spec(0, 1), _part_spec(1, 1)],
    out_specs=_row_spec(D),
    out_shape=jax.ShapeDtypeStruct((N, D), jnp.float32),
)

_tc2 = pl.pallas_call(
    _tc2_body,
    grid=(N // RB,),
    in_specs=[_part_spec(0, D), _part_spec(1, D), _part_spec(0, 1),
              _part_spec(1, 1), _full_spec((1, D)), _full_spec((D, D))],
    out_specs=_row_spec(D),
    out_shape=jax.ShapeDtypeStruct((N, D), jnp.float32),
)

_tc3 = pl.pallas_call(
    _tc3_body,
    grid=(N // RB,),
    in_specs=[_part_spec(0, D), _part_spec(1, D), _part_spec(0, 1),
              _part_spec(1, 1), _full_spec((1, D)), _full_spec((D, NCLS)),
              _full_spec((1, NCLS))],
    out_specs=_row_spec(NCLS),
    out_shape=jax.ShapeDtypeStruct((N, NCLS), jnp.float32),
)


def kernel(x, edge_index, W1, b1, W2, b2, Wc, bc):
    ei = edge_index.astype(jnp.int32).reshape(2, NW, EPT_R)
    zrows = jnp.zeros((RPS_OUT, D), jnp.float32)

    dp = _deg_call()(ei).reshape(NC, NPAD, 1)
    h1p = _tc1(x, W1, dp, dp)
    p1 = _agg_call()(h1p, ei, zrows)
    h2p = _tc2(p1, p1, dp, dp, b1.reshape(1, D), W2)
    p2 = _agg_call()(h2p, ei, zrows)
    out = _tc3(p2, p2, dp, dp, b2.reshape(1, D), Wc, bc.reshape(1, NCLS))
    return out


# speedup vs baseline: 1.0971x; 1.0971x over previous
import functools

import jax
import jax.numpy as jnp
from jax import lax
from jax.experimental import pallas as pl
from jax.experimental.pallas import tpu as pltpu
from jax.experimental.pallas import tpu_sc as plsc

N = 10000
E = 320000
D = 128
NCLS = 10
NC, NS = 2, 16
NW = NC * NS
NPAD = 10240
B = 128
EPT_R = E // NW
CPT_F = EPT_R // B
TAIL = EPT_R - CPT_F * B
TAILPAD = B - TAIL
CPT = CPT_F + 1
RPS_INIT = 632
RPS_OUT = NPAD // NS

_MESH = dict(core_axis_name="c", subcore_axis_name="s",
             num_cores=NC, num_subcores=NS)


def _fill_idx(didx_v, dibuf, i):
    for j in range(B // 16):
        didx_v[pl.ds(j * 16, 16)] = dibuf[pl.ds(i * B + j * 16, 16)]


def _deg_body(ei_hbm, out_hbm, ones_v, zeros_v, didx0, didx1, dibuf, acc,
              semd0, semd1):
    c = lax.axis_index("c")
    s = lax.axis_index("s")
    wid = s * NC + c
    base = wid * EPT_R
    for i in range(B // 16):
        ones_v[pl.ds(i * 16, 16)] = jnp.full((16,), 1.0, jnp.float32)
        zeros_v[pl.ds(i * 16, 16)] = jnp.zeros((16,), jnp.float32)
    for i in range(RPS_OUT // B):
        pltpu.sync_copy(zeros_v, acc.at[pl.ds(s * RPS_OUT + i * B, B)])
    pltpu.sync_copy(ei_hbm.at[1, wid], dibuf)
    plsc.subcore_barrier()

    def body(k, carry):
        i0 = 2 * k
        i1 = i0 + 1
        _fill_idx(didx0, dibuf, i0)
        pltpu.async_copy(ones_v, acc.at[didx0], semd0, add=True)
        _fill_idx(didx1, dibuf, i1)
        pltpu.async_copy(ones_v, acc.at[didx1], semd1, add=True)
        pltpu.make_async_copy(ones_v, acc.at[didx0], semd0).wait()
        pltpu.make_async_copy(ones_v, acc.at[didx1], semd1).wait()
        return carry

    lax.fori_loop(0, (CPT - 1) // 2, body, 0)
    didx0[pl.ds(0, 16)] = dibuf[pl.ds(EPT_R - TAIL, 16)]
    for j in range(1, B // 16):
        didx0[pl.ds(j * 16, 16)] = (
            lax.iota(jnp.int32, 16) + (N + (j - 1) * 16))
    pltpu.sync_copy(ones_v, acc.at[didx0], add=True)
    plsc.subcore_barrier()
    pltpu.sync_copy(acc.at[pl.ds(s * RPS_OUT, RPS_OUT)],
                    out_hbm.at[c, pl.ds(s * RPS_OUT, RPS_OUT)])


@functools.cache
def _deg_call():
    return pl.kernel(
        _deg_body,
        out_type=jax.ShapeDtypeStruct((NC, NPAD), jnp.float32),
        mesh=plsc.VectorSubcoreMesh(**_MESH),
        scratch_types=[
            pltpu.VMEM((B,), jnp.float32),
            pltpu.VMEM((B,), jnp.float32),
            pltpu.VMEM((B,), jnp.int32),
            pltpu.VMEM((B,), jnp.int32),
            pltpu.VMEM((EPT_R,), jnp.int32),
            pltpu.VMEM_SHARED((NPAD,), jnp.float32),
            pltpu.SemaphoreType.DMA,
            pltpu.SemaphoreType.DMA,
        ],
    )


def _agg_body(hp_hbm, ei_hbm, zeros_hbm, out_hbm,
              sibuf, sidx_t, didx0, didx1, didx_t, didx_t16, rows0, rows1,
              acc, semg0, semg1, semd0, semd1):
    c = lax.axis_index("c")
    s = lax.axis_index("s")
    wid = s * NC + c
    base = wid * EPT_R

    @pl.when(c == 0)
    def _():
        b0 = jnp.minimum(s * RPS_INIT, N - RPS_INIT)
        pltpu.sync_copy(hp_hbm.at[pl.ds(b0, RPS_INIT)],
                        acc.at[pl.ds(b0, RPS_INIT)])

    @pl.when(c == 1)
    def _():
        pltpu.sync_copy(zeros_hbm, acc.at[pl.ds(s * RPS_OUT, RPS_OUT)])

    pltpu.sync_copy(ei_hbm.at[0, wid], sibuf)
    sidx_t[pl.ds(0, 16)] = sibuf[pl.ds(EPT_R - TAIL, 16)]
    for j in range(1, B // 16):
        sidx_t[pl.ds(j * 16, 16)] = (
            (lax.iota(jnp.int32, 16) + j * 16) * 89)
    for j in range(1, B // 16):
        didx_t[pl.ds(j * 16, 16)] = (
            lax.iota(jnp.int32, 16) + (N + (j - 1) * 16))
    plsc.subcore_barrier()

    def gsrc(i):
        return hp_hbm.at[sibuf.at[pl.ds(i * B, B)]]

    def dsrc(i):
        return ei_hbm.at[1, wid, pl.ds(i * B, B)]

    tail_dsrc = ei_hbm.at[1, wid, pl.ds(EPT_R - TAIL, TAIL)]

    pltpu.async_copy(dsrc(0), didx0, semd0)
    pltpu.async_copy(gsrc(0), rows0, semg0)

    def body(k, carry):
        i0 = 2 * k
        i1 = i0 + 1
        i2 = i0 + 2
        pltpu.async_copy(dsrc(i1), didx1, semd1)
        pltpu.async_copy(gsrc(i1), rows1, semg1)
        pltpu.make_async_copy(gsrc(i0), rows0, semg0).wait()
        pltpu.make_async_copy(dsrc(i0), didx0, semd0).wait()
        pltpu.sync_copy(rows0, acc.at[didx0], add=True)
        pltpu.async_copy(dsrc(i2), didx0, semd0)
        pltpu.async_copy(gsrc(i2), rows0, semg0)
        pltpu.make_async_copy(gsrc(i1), rows1, semg1).wait()
        pltpu.make_async_copy(dsrc(i1), didx1, semd1).wait()
        pltpu.sync_copy(rows1, acc.at[didx1], add=True)
        return carry

    lax.fori_loop(0, (CPT_F - 2) // 2, body, 0)
    pltpu.async_copy(dsrc(CPT_F - 1), didx1, semd1)
    pltpu.async_copy(gsrc(CPT_F - 1), rows1, semg1)
    pltpu.make_async_copy(gsrc(CPT_F - 2), rows0, semg0).wait()
    pltpu.make_async_copy(dsrc(CPT_F - 2), didx0, semd0).wait()
    pltpu.sync_copy(rows0, acc.at[didx0], add=True)
    pltpu.async_copy(hp_hbm.at[sidx_t], rows0, semg0)
    pltpu.async_copy(tail_dsrc, didx_t16, semd0)
    pltpu.make_async_copy(gsrc(CPT_F - 1), rows1, semg1).wait()
    pltpu.make_async_copy(dsrc(CPT_F - 1), didx1, semd1).wait()
    pltpu.sync_copy(rows1, acc.at[didx1], add=True)
    pltpu.make_async_copy(hp_hbm.at[sidx_t], rows0, semg0).wait()
    pltpu.make_async_copy(tail_dsrc, didx_t16, semd0).wait()
    didx_t[pl.ds(0, TAIL)] = didx_t16[...]
    pltpu.sync_copy(rows0, acc.at[didx_t], add=True)
    plsc.subcore_barrier()
    pltpu.sync_copy(acc.at[pl.ds(s * RPS_OUT, RPS_OUT)],
                    out_hbm.at[c, pl.ds(s * RPS_OUT, RPS_OUT)])


@functools.cache
def _agg_call():
    return pl.kernel(
        _agg_body,
        out_type=jax.ShapeDtypeStruct((NC, NPAD, D), jnp.float32),
        mesh=plsc.VectorSubcoreMesh(**_MESH),
        scratch_types=[
            pltpu.VMEM((EPT_R,), jnp.int32),
            pltpu.VMEM((B,), jnp.int32),
            pltpu.VMEM((B,), jnp.int32),
            pltpu.VMEM((B,), jnp.int32),
            pltpu.VMEM((B,), jnp.int32),
            pltpu.VMEM((TAIL,), jnp.int32),
            pltpu.VMEM((B, D), jnp.float32),
            pltpu.VMEM((B, D), jnp.float32),
            pltpu.VMEM_SHARED((NPAD, D), jnp.float32),
            pltpu.SemaphoreType.DMA,
            pltpu.SemaphoreType.DMA,
            pltpu.SemaphoreType.DMA,
            pltpu.SemaphoreType.DMA,
        ],
    )


RB = 2000


def _dis(d0_ref, d1_ref):
    return lax.rsqrt(d0_ref[0] + d1_ref[0] + 1.0)


def _tc1_body(x_ref, w_ref, d0_ref, d1_ref, o_ref):
    h = jnp.dot(x_ref[...], w_ref[...], preferred_element_type=jnp.float32)
    o_ref[...] = h * _dis(d0_ref, d1_ref)


def _tc2_body(a0_ref, a1_ref, d0_ref, d1_ref, b_ref, w_ref, o_ref):
    dis = _dis(d0_ref, d1_ref)
    h1 = jnp.maximum((a0_ref[0] + a1_ref[0]) * dis + b_ref[...], 0.0)
    o_ref[...] = jnp.dot(h1, w_ref[...], preferred_element_type=jnp.float32) * dis


def _tc3_body(a0_ref, a1_ref, d0_ref, d1_ref, b_ref, w_ref, bc_ref, o_ref):
    dis = _dis(d0_ref, d1_ref)
    h2 = jnp.maximum((a0_ref[0] + a1_ref[0]) * dis + b_ref[...], 0.0)
    o_ref[...] = (jnp.dot(h2, w_ref[...], preferred_element_type=jnp.float32)
                  + bc_ref[...])


def _pad_cols(a, w):
    return jnp.concatenate(
        [a, jnp.zeros((a.shape[0], w - a.shape[1]), a.dtype)], axis=1)


def _row_spec(width):
    return pl.BlockSpec((RB, width), lambda i: (i, 0))


def _part_spec(core, width):
    return pl.BlockSpec((1, RB, width), lambda i, _c=core: (_c, i, 0))


def _full_spec(shape):
    return pl.BlockSpec(shape, lambda i: tuple(0 for _ in shape))


_tc1 = pl.pallas_call(
    _tc1_body,
    grid=(N // RB,),
    in_specs=[_row_spec(D), _full_spec((D, D)),
              _part_spec(0, 1), _part_spec(1, 1)],
    out_specs=_row_spec(D),
    out_shape=jax.ShapeDtypeStruct((N, D), jnp.float32),
)

_tc2 = pl.pallas_call(
    _tc2_body,
    grid=(N // RB,),
    in_specs=[_part_spec(0, D), _part_spec(1, D), _part_spec(0, 1),
              _part_spec(1, 1), _full_spec((1, D)), _full_spec((D, D))],
    out_specs=_row_spec(D),
    out_shape=jax.ShapeDtypeStruct((N, D), jnp.float32),
)

_tc3 = pl.pallas_call(
    _tc3_body,
    grid=(N // RB,),
    in_specs=[_part_spec(0, D), _part_spec(1, D), _part_spec(0, 1),
              _part_spec(1, 1), _full_spec((1, D)), _full_spec((D, D)),
              _full_spec((1, D))],
    out_specs=_row_spec(D),
    out_shape=jax.ShapeDtypeStruct((N, D), jnp.float32),
)


def kernel(x, edge_index, W1, b1, W2, b2, Wc, bc):
    ei = edge_index.astype(jnp.int32).reshape(2, NW, EPT_R)
    zrows = jnp.zeros((RPS_OUT, D), jnp.float32)

    dp = _deg_call()(ei).reshape(NC, NPAD, 1)
    h1p = _tc1(x, W1, dp, dp)
    p1 = _agg_call()(h1p, ei, zrows)
    h2p = _tc2(p1, p1, dp, dp, b1.reshape(1, D), W2)
    p2 = _agg_call()(h2p, ei, zrows)
    out = _tc3(p2, p2, dp, dp, b2.reshape(1, D), _pad_cols(Wc, D),
               _pad_cols(bc.reshape(1, NCLS), D))
    return out[:, :NCLS]
